# baseline (speedup 1.0000x reference)
import functools
import math

import jax
import jax.numpy as jnp
from jax import lax
from jax.experimental import pallas as pl
from jax.experimental.pallas import tpu as pltpu
from jax.experimental.pallas import tpu_sc as plsc

D_MODEL = 64
VOCAB_ROWS = 100001
SCALE = math.sqrt(D_MODEL)

NUM_CORES = 2
NUM_SUBCORES = 16
NUM_WORKERS = NUM_CORES * NUM_SUBCORES

CHUNK = 100
NBUF = 8


_PRESCALE_BLK = 8192


def _prescale_body(x_ref, o_ref):
    row0 = pl.program_id(0) * _PRESCALE_BLK
    rid = lax.broadcasted_iota(jnp.int32, x_ref.shape, 0) + row0
    o_ref[...] = x_ref[...] * jnp.where(rid == 0, 0.0, jnp.float32(SCALE))


def _prescale_table(table):
    nblk = pl.cdiv(table.shape[0], _PRESCALE_BLK)
    return pl.pallas_call(
        _prescale_body,
        grid=(nblk,),
        in_specs=[pl.BlockSpec((_PRESCALE_BLK, D_MODEL), lambda i: (i, 0))],
        out_specs=pl.BlockSpec((_PRESCALE_BLK, D_MODEL), lambda i: (i, 0)),
        out_shape=jax.ShapeDtypeStruct(table.shape, jnp.float32),
    )(table)


def _make_gather(batch, seqlen):
    assert seqlen == 2 * CHUNK
    num_tokens = batch * seqlen
    per_worker = num_tokens // NUM_WORKERS
    batches_per_worker = per_worker // seqlen
    assert batches_per_worker * seqlen == per_worker
    n_chunks = per_worker // CHUNK
    assert n_chunks % NBUF == 0
    n_groups = n_chunks // NBUF

    mesh = plsc.VectorSubcoreMesh(
        core_axis_name="c", subcore_axis_name="s",
        num_cores=NUM_CORES, num_subcores=NUM_SUBCORES)

    @functools.partial(
        pl.kernel,
        out_type=jax.ShapeDtypeStruct((batch, seqlen, D_MODEL), jnp.float32),
        mesh=mesh,
        compiler_params=pltpu.CompilerParams(use_tc_tiling_on_sc=False),
        scratch_types=[
            pltpu.VMEM((n_chunks, CHUNK), jnp.int32),
            pltpu.VMEM((NBUF, CHUNK, D_MODEL), jnp.float32),
            pltpu.SemaphoreType.DMA,
            pltpu.SemaphoreType.DMA,
        ],
    )
    def gather_kernel(tok_hbm, table_hbm, out_hbm, idx_v, rows_v, gsem, osem):
        wid = lax.axis_index("s") * NUM_CORES + lax.axis_index("c")
        batch_base = wid * batches_per_worker
        pltpu.sync_copy(tok_hbm.at[pl.ds(wid * n_chunks, n_chunks)], idx_v)

        def group(g, _):
            j0 = g * NBUF
            gathers = []
            for b in range(NBUF):
                dma = pltpu.make_async_copy(
                    table_hbm.at[idx_v.at[j0 + b]], rows_v.at[b], gsem)
                dma.start()
                gathers.append(dma)
            outs = []
            for b in range(NBUF):
                j = j0 + b
                gathers[b].wait()
                dma = pltpu.make_async_copy(
                    rows_v.at[b],
                    out_hbm.at[batch_base + j // 2, pl.ds((j % 2) * CHUNK, CHUNK)],
                    osem)
                dma.start()
                outs.append(dma)
            for b in range(NBUF):
                outs[b].wait()
            return 0

        lax.fori_loop(0, n_groups, group, 0)

    return gather_kernel


def kernel(tokens, table):
    batch, seqlen = tokens.shape
    num_tokens = batch * seqlen
    idx = tokens.reshape(num_tokens // CHUNK, CHUNK).astype(jnp.int32)
    scaled = _prescale_table(table)
    return _make_gather(batch, seqlen)(idx, scaled)


# speedup vs baseline: 1.0623x; 1.0623x over previous
import functools
import math

import jax
import jax.numpy as jnp
from jax import lax
from jax.experimental import pallas as pl
from jax.experimental.pallas import tpu as pltpu
from jax.experimental.pallas import tpu_sc as plsc

D_MODEL = 64
SCALE = math.sqrt(D_MODEL)

NUM_CORES = 2
NUM_SUBCORES = 16
NUM_WORKERS = NUM_CORES * NUM_SUBCORES

CHUNK = 128
NBUF = 8

B_BLK = 512
Q_BLK = 4


def _make_gather(num_tokens):
    assert num_tokens % (NUM_WORKERS * CHUNK) == 0
    per_worker = num_tokens // NUM_WORKERS
    n_chunks = per_worker // CHUNK
    assert n_chunks % NBUF == 0
    n_groups = n_chunks // NBUF

    mesh = plsc.VectorSubcoreMesh(
        core_axis_name="c", subcore_axis_name="s",
        num_cores=NUM_CORES, num_subcores=NUM_SUBCORES)

    @functools.partial(
        pl.kernel,
        out_type=jax.ShapeDtypeStruct((num_tokens, D_MODEL), jnp.float32),
        mesh=mesh,
        compiler_params=pltpu.CompilerParams(use_tc_tiling_on_sc=False),
        scratch_types=[
            pltpu.VMEM((n_chunks, CHUNK), jnp.int32),
            pltpu.VMEM((NBUF, CHUNK, D_MODEL), jnp.float32),
            pltpu.SemaphoreType.DMA,
            pltpu.SemaphoreType.DMA,
        ],
    )
    def gather_kernel(tok_hbm, table_hbm, out_hbm, idx_v, rows_v, gsem, osem):
        wid = lax.axis_index("s") * NUM_CORES + lax.axis_index("c")
        row_base = wid * per_worker
        pltpu.sync_copy(tok_hbm.at[pl.ds(wid * n_chunks, n_chunks)], idx_v)

        def group(g, _):
            j0 = g * NBUF
            gathers = []
            for b in range(NBUF):
                dma = pltpu.make_async_copy(
                    table_hbm.at[idx_v.at[j0 + b]], rows_v.at[b], gsem)
                dma.start()
                gathers.append(dma)
            outs = []
            for b in range(NBUF):
                gathers[b].wait()
                dma = pltpu.make_async_copy(
                    rows_v.at[b],
                    out_hbm.at[pl.ds(row_base + (j0 + b) * CHUNK, CHUNK)],
                    osem)
                dma.start()
                outs.append(dma)
            for b in range(NBUF):
                outs[b].wait()
            return 0

        lax.fori_loop(0, n_groups, group, 0)

    return gather_kernel


def _finisher_body(g_ref, s_ref, out_ref):
    for qi in range(Q_BLK):
        for si in range(2):
            xs = g_ref[qi, :, si * D_MODEL:(si + 1) * D_MODEL]
            scale = s_ref[2 * qi + si, :]
            out_ref[2 * qi + si, :, :] = xs.T * scale[None, :]


def _finish(gathered, scale_sb, batch, seqlen):
    g4 = gathered.reshape(seqlen // 2, batch, 2 * D_MODEL)
    out_t = pl.pallas_call(
        _finisher_body,
        grid=(seqlen // (2 * Q_BLK), batch // B_BLK),
        in_specs=[
            pl.BlockSpec((Q_BLK, B_BLK, 2 * D_MODEL), lambda q, i: (q, i, 0)),
            pl.BlockSpec((2 * Q_BLK, B_BLK), lambda q, i: (q, i)),
        ],
        out_specs=pl.BlockSpec((2 * Q_BLK, D_MODEL, B_BLK), lambda q, i: (q, 0, i)),
        out_shape=jax.ShapeDtypeStruct((seqlen, D_MODEL, batch), jnp.float32),
    )(g4, scale_sb)
    return out_t.transpose(2, 0, 1)


def kernel(tokens, table):
    batch, seqlen = tokens.shape
    num_tokens = batch * seqlen
    tokens = tokens.astype(jnp.int32)
    idx = (tokens.reshape(batch, seqlen // 2, 2)
           .transpose(1, 0, 2)
           .reshape(num_tokens // CHUNK, CHUNK))
    scale_sb = jnp.where(tokens.T == 0, 0.0, jnp.float32(SCALE))
    gathered = _make_gather(num_tokens)(idx, table)
    return _finish(gathered, scale_sb, batch, seqlen)


# speedup vs baseline: 1.5638x; 1.4722x over previous
import functools
import math

import jax
import jax.numpy as jnp
from jax import lax
from jax.experimental import pallas as pl
from jax.experimental.pallas import tpu as pltpu
from jax.experimental.pallas import tpu_sc as plsc

D_MODEL = 64
SCALE = math.sqrt(D_MODEL)

NUM_CORES = 2
NUM_SUBCORES = 16
NUM_WORKERS = NUM_CORES * NUM_SUBCORES

CHUNK = 128
NBUF = 8

B_BLK = 512
Q_BLK = 4


def _make_gather(batch, seqlen):
    num_tokens = batch * seqlen
    per_worker = num_tokens // NUM_WORKERS
    bw = batch // NUM_WORKERS
    assert bw == CHUNK and seqlen % 2 == 0
    n_chunks = per_worker // CHUNK
    assert NBUF % 2 == 0 and n_chunks % NBUF == 0
    n_groups = n_chunks // NBUF

    mesh = plsc.VectorSubcoreMesh(
        core_axis_name="c", subcore_axis_name="s",
        num_cores=NUM_CORES, num_subcores=NUM_SUBCORES)

    @functools.partial(
        pl.kernel,
        out_type=jax.ShapeDtypeStruct((num_tokens, D_MODEL), jnp.float32),
        mesh=mesh,
        compiler_params=pltpu.CompilerParams(
            use_tc_tiling_on_sc=False, needs_layout_passes=False),
        scratch_types=[
            pltpu.VMEM((seqlen, CHUNK), jnp.int32),
            pltpu.VMEM((NBUF, CHUNK), jnp.int32),
            pltpu.VMEM((NBUF, CHUNK, D_MODEL), jnp.float32),
            pltpu.SemaphoreType.DMA,
            pltpu.SemaphoreType.DMA,
        ],
    )
    def gather_kernel(tokt_hbm, table_hbm, out_hbm, tok_v, idx_v, rows_v,
                      gsem, osem):
        wid = lax.axis_index("s") * NUM_CORES + lax.axis_index("c")
        pltpu.sync_copy(tokt_hbm.at[:, pl.ds(wid * bw, bw)], tok_v)
        ev2 = 2 * lax.iota(jnp.int32, 16)

        def group(g, _):
            q0 = g * (NBUF // 2)
            for b in range(NBUF):
                q = q0 + b // 2
                h = b % 2
                for r in range(2):
                    row = 2 * q + r
                    for gg in range(4):
                        src = tok_v[row, pl.ds(h * 64 + gg * 16, 16)]
                        plsc.store_scatter(
                            idx_v.at[b], [ev2 + (32 * gg + r)], src)
            gathers = []
            for b in range(NBUF):
                dma = pltpu.make_async_copy(
                    table_hbm.at[idx_v.at[b]], rows_v.at[b], gsem)
                dma.start()
                gathers.append(dma)
            outs = []
            for b in range(NBUF):
                j = g * NBUF + b
                q = q0 + b // 2
                h = b % 2
                gathers[b].wait()
                dma = pltpu.make_async_copy(
                    rows_v.at[b],
                    out_hbm.at[pl.ds(
                        (q * batch + wid * bw + h * 64) * 2, CHUNK)],
                    osem)
                dma.start()
                outs.append(dma)
            for b in range(NBUF):
                outs[b].wait()
            return 0

        lax.fori_loop(0, n_groups, group, 0)

    return gather_kernel


def _finisher_body(g_ref, s_ref, out_ref):
    for qi in range(Q_BLK):
        for si in range(2):
            xs = g_ref[qi, :, si * D_MODEL:(si + 1) * D_MODEL]
            scale = s_ref[2 * qi + si, :]
            out_ref[2 * qi + si, :, :] = xs.T * scale[None, :]


def _finish(gathered, scale_sb, batch, seqlen):
    g4 = gathered.reshape(seqlen // 2, batch, 2 * D_MODEL)
    out_t = pl.pallas_call(
        _finisher_body,
        grid=(seqlen // (2 * Q_BLK), batch // B_BLK),
        in_specs=[
            pl.BlockSpec((Q_BLK, B_BLK, 2 * D_MODEL), lambda q, i: (q, i, 0)),
            pl.BlockSpec((2 * Q_BLK, B_BLK), lambda q, i: (q, i)),
        ],
        out_specs=pl.BlockSpec((2 * Q_BLK, D_MODEL, B_BLK), lambda q, i: (q, 0, i)),
        out_shape=jax.ShapeDtypeStruct((seqlen, D_MODEL, batch), jnp.float32),
    )(g4, scale_sb)
    return out_t.transpose(2, 0, 1)


def kernel(tokens, table):
    batch, seqlen = tokens.shape
    tokens_t = tokens.T.astype(jnp.int32)
    scale_sb = jnp.where(tokens_t == 0, 0.0, jnp.float32(SCALE))
    gathered = _make_gather(batch, seqlen)(tokens_t, table)
    return _finish(gathered, scale_sb, batch, seqlen)


# speedup vs baseline: 1.7287x; 1.1054x over previous
import functools
import math

import jax
import jax.numpy as jnp
from jax import lax
from jax.experimental import pallas as pl
from jax.experimental.pallas import tpu as pltpu
from jax.experimental.pallas import tpu_sc as plsc

D_MODEL = 64
SCALE = math.sqrt(D_MODEL)

NUM_CORES = 2
NUM_SUBCORES = 16
NUM_WORKERS = NUM_CORES * NUM_SUBCORES

CHUNK = 128
NBUF = 8

B_BLK = 512
Q_BLK = 4


def _make_gather(batch, seqlen):
    num_tokens = batch * seqlen
    per_worker = num_tokens // NUM_WORKERS
    bw = batch // NUM_WORKERS
    assert bw == CHUNK and seqlen % 2 == 0
    n_chunks = per_worker // CHUNK
    assert NBUF % 2 == 0 and n_chunks % NBUF == 0
    n_groups = n_chunks // NBUF

    mesh = plsc.VectorSubcoreMesh(
        core_axis_name="c", subcore_axis_name="s",
        num_cores=NUM_CORES, num_subcores=NUM_SUBCORES)

    @functools.partial(
        pl.kernel,
        out_type=jax.ShapeDtypeStruct((num_tokens, D_MODEL), jnp.float32),
        mesh=mesh,
        compiler_params=pltpu.CompilerParams(
            use_tc_tiling_on_sc=False, needs_layout_passes=False),
        scratch_types=[
            pltpu.VMEM((seqlen, CHUNK), jnp.int32),
            pltpu.VMEM((NBUF, CHUNK), jnp.int32),
            pltpu.VMEM((NBUF, CHUNK, D_MODEL), jnp.float32),
            pltpu.SemaphoreType.DMA,
            pltpu.SemaphoreType.DMA,
        ],
    )
    def gather_kernel(tokt_hbm, table_hbm, out_hbm, tok_v, idx_v, rows_v,
                      gsem, osem):
        wid = lax.axis_index("s") * NUM_CORES + lax.axis_index("c")
        pltpu.sync_copy(tokt_hbm.at[:, pl.ds(wid * bw, bw)], tok_v)
        ev2 = 2 * lax.iota(jnp.int32, 16)

        def group(g, _):
            q0 = g * (NBUF // 2)
            for b in range(NBUF):
                q = q0 + b // 2
                h = b % 2
                for r in range(2):
                    row = 2 * q + r
                    for gg in range(4):
                        src = tok_v[row, pl.ds(h * 64 + gg * 16, 16)]
                        plsc.store_scatter(
                            idx_v.at[b], [ev2 + (32 * gg + r)], src)
            gathers = []
            for b in range(NBUF):
                dma = pltpu.make_async_copy(
                    table_hbm.at[idx_v.at[b]], rows_v.at[b], gsem)
                dma.start()
                gathers.append(dma)
            outs = []
            for b in range(NBUF):
                j = g * NBUF + b
                q = q0 + b // 2
                h = b % 2
                gathers[b].wait()
                dma = pltpu.make_async_copy(
                    rows_v.at[b],
                    out_hbm.at[pl.ds(
                        (q * batch + wid * bw + h * 64) * 2, CHUNK)],
                    osem)
                dma.start()
                outs.append(dma)
            for b in range(NBUF):
                outs[b].wait()
            return 0

        lax.fori_loop(0, n_groups, group, 0)

    return gather_kernel


def _finisher_body(g_ref, s_ref, out_ref):
    for qi in range(Q_BLK):
        yt = g_ref[qi].T
        for si in range(2):
            scale = s_ref[2 * qi + si, :]
            out_ref[2 * qi + si, :, :] = (
                yt[si * D_MODEL:(si + 1) * D_MODEL, :] * scale[None, :])


def _finish(gathered, scale_sb, batch, seqlen):
    g4 = gathered.reshape(seqlen // 2, batch, 2 * D_MODEL)
    out_t = pl.pallas_call(
        _finisher_body,
        grid=(seqlen // (2 * Q_BLK), batch // B_BLK),
        in_specs=[
            pl.BlockSpec((Q_BLK, B_BLK, 2 * D_MODEL), lambda q, i: (q, i, 0)),
            pl.BlockSpec((2 * Q_BLK, B_BLK), lambda q, i: (q, i)),
        ],
        out_specs=pl.BlockSpec((2 * Q_BLK, D_MODEL, B_BLK), lambda q, i: (q, 0, i)),
        out_shape=jax.ShapeDtypeStruct((seqlen, D_MODEL, batch), jnp.float32),
    )(g4, scale_sb)
    return out_t.transpose(2, 0, 1)


def kernel(tokens, table):
    batch, seqlen = tokens.shape
    tokens_t = tokens.T.astype(jnp.int32)
    scale_sb = jnp.where(tokens_t == 0, 0.0, jnp.float32(SCALE))
    gathered = _make_gather(batch, seqlen)(tokens_t, table)
    return _finish(gathered, scale_sb, batch, seqlen)


# speedup vs baseline: 1.8132x; 1.0489x over previous
import functools
import math

import jax
import jax.numpy as jnp
from jax import lax
from jax.experimental import pallas as pl
from jax.experimental.pallas import tpu as pltpu
from jax.experimental.pallas import tpu_sc as plsc

D_MODEL = 64
SCALE = math.sqrt(D_MODEL)

NUM_CORES = 2
NUM_SUBCORES = 16
NUM_WORKERS = NUM_CORES * NUM_SUBCORES

CHUNK = 128
NBUF = 8

B_BLK = 512
Q_BLK = 4


def _make_gather(batch, seqlen):
    num_tokens = batch * seqlen
    per_worker = num_tokens // NUM_WORKERS
    bw = batch // NUM_WORKERS
    assert bw == CHUNK and seqlen % 2 == 0
    n_chunks = per_worker // CHUNK
    assert NBUF % 2 == 0 and n_chunks % NBUF == 0
    n_groups = n_chunks // NBUF

    mesh = plsc.VectorSubcoreMesh(
        core_axis_name="c", subcore_axis_name="s",
        num_cores=NUM_CORES, num_subcores=NUM_SUBCORES)

    @functools.partial(
        pl.kernel,
        out_type=jax.ShapeDtypeStruct((num_tokens, D_MODEL), jnp.float32),
        mesh=mesh,
        compiler_params=pltpu.CompilerParams(
            use_tc_tiling_on_sc=False, needs_layout_passes=False),
        scratch_types=[
            pltpu.VMEM((seqlen, CHUNK), jnp.int32),
            pltpu.VMEM((NBUF, CHUNK), jnp.int32),
            pltpu.VMEM((NBUF, CHUNK, D_MODEL), jnp.float32),
            pltpu.SemaphoreType.DMA,
            pltpu.SemaphoreType.DMA,
        ],
    )
    def gather_kernel(tokt_hbm, table_hbm, out_hbm, tok_v, idx_v, rows_v,
                      gsem, osem):
        wid = lax.axis_index("s") * NUM_CORES + lax.axis_index("c")
        pltpu.sync_copy(tokt_hbm.at[:, pl.ds(wid * bw, bw)], tok_v)
        ev2 = 2 * lax.iota(jnp.int32, 16)

        def group(g, _):
            q0 = g * (NBUF // 2)
            for b in range(NBUF):
                q = q0 + b // 2
                h = b % 2
                for r in range(2):
                    row = 2 * q + r
                    for gg in range(4):
                        src = tok_v[row, pl.ds(h * 64 + gg * 16, 16)]
                        plsc.store_scatter(
                            idx_v.at[b], [ev2 + (32 * gg + r)], src)
            gathers = []
            for b in range(NBUF):
                dma = pltpu.make_async_copy(
                    table_hbm.at[idx_v.at[b]], rows_v.at[b], gsem)
                dma.start()
                gathers.append(dma)
            outs = []
            for b in range(NBUF):
                j = g * NBUF + b
                q = q0 + b // 2
                h = b % 2
                gathers[b].wait()
                dma = pltpu.make_async_copy(
                    rows_v.at[b],
                    out_hbm.at[pl.ds(
                        (q * batch + wid * bw + h * 64) * 2, CHUNK)],
                    osem)
                dma.start()
                outs.append(dma)
            for b in range(NBUF):
                outs[b].wait()
            return 0

        lax.fori_loop(0, n_groups, group, 0)

    return gather_kernel


def _finisher_body(g_ref, s_ref, out_ref):
    for qi in range(Q_BLK):
        yt = g_ref[qi].T
        for si in range(2):
            scale = s_ref[2 * qi + si, :]
            out_ref[2 * qi + si, :, :] = (
                yt[si * D_MODEL:(si + 1) * D_MODEL, :] * scale[None, :])


def _finish(gathered, scale_sb, batch, seqlen, s_off, total_seqlen, donated):
    g4 = gathered.reshape(seqlen // 2, batch, 2 * D_MODEL)
    qb_off = s_off // (2 * Q_BLK)
    body = _finisher_body if donated is None else (
        lambda g_ref, s_ref, _, out_ref: _finisher_body(g_ref, s_ref, out_ref))
    in_specs = [
        pl.BlockSpec((Q_BLK, B_BLK, 2 * D_MODEL), lambda q, i: (q, i, 0)),
        pl.BlockSpec((2 * Q_BLK, B_BLK), lambda q, i: (q, i)),
    ]
    args = [g4, scale_sb]
    kwargs = {}
    if donated is not None:
        in_specs.append(pl.BlockSpec(memory_space=pl.ANY))
        args.append(donated)
        kwargs["input_output_aliases"] = {2: 0}
    return pl.pallas_call(
        body,
        grid=(seqlen // (2 * Q_BLK), batch // B_BLK),
        in_specs=in_specs,
        out_specs=pl.BlockSpec(
            (2 * Q_BLK, D_MODEL, B_BLK), lambda q, i: (q + qb_off, 0, i)),
        out_shape=jax.ShapeDtypeStruct((total_seqlen, D_MODEL, batch),
                                       jnp.float32),
        **kwargs,
    )(*args)


def kernel(tokens, table):
    batch, seqlen = tokens.shape
    tokens_t = tokens.T.astype(jnp.int32)
    scale_sb = jnp.where(tokens_t == 0, 0.0, jnp.float32(SCALE))
    s_split = 96
    out_t = None
    for s0, s1 in ((0, s_split), (s_split, seqlen)):
        sl = s1 - s0
        g = _make_gather(batch, sl)(tokens_t[s0:s1], table)
        out_t = _finish(g, scale_sb[s0:s1], batch, sl, s0, seqlen, out_t)
    return out_t.transpose(2, 0, 1)


# speedup vs baseline: 1.8208x; 1.0041x over previous
import functools
import math

import jax
import jax.numpy as jnp
from jax import lax
from jax.experimental import pallas as pl
from jax.experimental.pallas import tpu as pltpu
from jax.experimental.pallas import tpu_sc as plsc

D_MODEL = 64
SCALE = math.sqrt(D_MODEL)

NUM_CORES = 2
NUM_SUBCORES = 16
NUM_WORKERS = NUM_CORES * NUM_SUBCORES

CHUNK = 128
NBUF = 8

B_BLK = 512
Q_BLK = 4


def _make_gather(batch, seqlen):
    num_tokens = batch * seqlen
    per_worker = num_tokens // NUM_WORKERS
    bw = batch // NUM_WORKERS
    assert bw == CHUNK and seqlen % 2 == 0
    n_chunks = per_worker // CHUNK
    assert NBUF % 2 == 0 and n_chunks % NBUF == 0
    n_groups = n_chunks // NBUF

    mesh = plsc.VectorSubcoreMesh(
        core_axis_name="c", subcore_axis_name="s",
        num_cores=NUM_CORES, num_subcores=NUM_SUBCORES)

    @functools.partial(
        pl.kernel,
        out_type=jax.ShapeDtypeStruct((num_tokens, D_MODEL), jnp.float32),
        mesh=mesh,
        compiler_params=pltpu.CompilerParams(
            use_tc_tiling_on_sc=False, needs_layout_passes=False),
        scratch_types=[
            pltpu.VMEM((seqlen, CHUNK), jnp.int32),
            pltpu.VMEM((NBUF, CHUNK), jnp.int32),
            pltpu.VMEM((NBUF, CHUNK, D_MODEL), jnp.float32),
            pltpu.SemaphoreType.DMA,
            pltpu.SemaphoreType.DMA,
        ],
    )
    def gather_kernel(tokt_hbm, table_hbm, out_hbm, tok_v, idx_v, rows_v,
                      gsem, osem):
        wid = lax.axis_index("s") * NUM_CORES + lax.axis_index("c")
        pltpu.sync_copy(tokt_hbm.at[:, pl.ds(wid * bw, bw)], tok_v)
        ev2 = 2 * lax.iota(jnp.int32, 16)

        def group(g, _):
            q0 = g * (NBUF // 2)
            for b in range(NBUF):
                q = q0 + b // 2
                h = b % 2
                for r in range(2):
                    row = 2 * q + r
                    for gg in range(4):
                        src = tok_v[row, pl.ds(h * 64 + gg * 16, 16)]
                        plsc.store_scatter(
                            idx_v.at[b], [ev2 + (32 * gg + r)], src)
            gathers = []
            for b in range(NBUF):
                dma = pltpu.make_async_copy(
                    table_hbm.at[idx_v.at[b]], rows_v.at[b], gsem)
                dma.start()
                gathers.append(dma)
            outs = []
            for b in range(NBUF):
                j = g * NBUF + b
                q = q0 + b // 2
                h = b % 2
                gathers[b].wait()
                dma = pltpu.make_async_copy(
                    rows_v.at[b],
                    out_hbm.at[pl.ds(
                        (q * batch + wid * bw + h * 64) * 2, CHUNK)],
                    osem)
                dma.start()
                outs.append(dma)
            for b in range(NBUF):
                outs[b].wait()
            return 0

        lax.fori_loop(0, n_groups, group, 0)

    return gather_kernel


def _finisher_body(g_ref, s_ref, out_ref):
    for qi in range(Q_BLK):
        yt = g_ref[qi].T
        for si in range(2):
            scale = s_ref[2 * qi + si, :]
            out_ref[2 * qi + si, :, :] = (
                yt[si * D_MODEL:(si + 1) * D_MODEL, :] * scale[None, :])


def _finish(gathered, scale_sb, batch, seqlen, s_off, total_seqlen, donated):
    g4 = gathered.reshape(seqlen // 2, batch, 2 * D_MODEL)
    qb_off = s_off // (2 * Q_BLK)
    body = _finisher_body if donated is None else (
        lambda g_ref, s_ref, _, out_ref: _finisher_body(g_ref, s_ref, out_ref))
    in_specs = [
        pl.BlockSpec((Q_BLK, B_BLK, 2 * D_MODEL), lambda q, i: (q, i, 0)),
        pl.BlockSpec((2 * Q_BLK, B_BLK), lambda q, i: (q, i)),
    ]
    args = [g4, scale_sb]
    kwargs = {}
    if donated is not None:
        in_specs.append(pl.BlockSpec(memory_space=pl.ANY))
        args.append(donated)
        kwargs["input_output_aliases"] = {2: 0}
    return pl.pallas_call(
        body,
        grid=(seqlen // (2 * Q_BLK), batch // B_BLK),
        in_specs=in_specs,
        out_specs=pl.BlockSpec(
            (2 * Q_BLK, D_MODEL, B_BLK), lambda q, i: (q + qb_off, 0, i)),
        out_shape=jax.ShapeDtypeStruct((total_seqlen, D_MODEL, batch),
                                       jnp.float32),
        **kwargs,
    )(*args)


def kernel(tokens, table):
    batch, seqlen = tokens.shape
    tokens_t = tokens.T.astype(jnp.int32)
    scale_sb = jnp.where(tokens_t == 0, 0.0, jnp.float32(SCALE))
    splits = (0, 48, 96, 144, seqlen)
    out_t = None
    for s0, s1 in zip(splits[:-1], splits[1:]):
        sl = s1 - s0
        g = _make_gather(batch, sl)(tokens_t[s0:s1], table)
        out_t = _finish(g, scale_sb[s0:s1], batch, sl, s0, seqlen, out_t)
    return out_t.transpose(2, 0, 1)
